# baseline (speedup 1.0000x reference)
import functools

import jax
import jax.numpy as jnp
from jax import lax
from jax.experimental import pallas as pl
from jax.experimental.pallas import tpu as pltpu
from jax.experimental.pallas import tpu_sc as plsc

LANES = 16
NC = 2
NS = 16
NW = NC * NS


def _sc_pool(B, L, V, D):
  b_per_w = B // NW
  n_d = D // LANES
  n_w = D // (2 * LANES)
  CHOFF, CHLEN = (0, 96), (96, L - 96)
  LTP = ((L + LANES - 1) // LANES) * LANES
  n_g = LTP // LANES
  mesh = plsc.VectorSubcoreMesh(core_axis_name="c", subcore_axis_name="s")

  def body(x_hbm, w_hbm, tab_hbm, out_hbm, idx_v, rows_v, wv_v, stage_v, sems):
    wid = lax.axis_index("s") * NC + lax.axis_index("c")
    base = wid * b_per_w
    pltpu.sync_copy(x_hbm.at[pl.ds(base, b_per_w)], idx_v)

    if LTP != L:
      pad_lo = (L // LANES) * LANES
      zvec = jnp.zeros((LANES,), jnp.int32)
      for p in range(2):
        wv_v[p, pl.ds(pad_lo, LANES)] = jnp.full((LANES,), -1e30, jnp.float32)
        for t in range(L, LTP):
          for k in range(n_w):
            rows_v[p, t, pl.ds(k * LANES, LANES)] = zvec

    def fire(r, p):
      for off, ln in zip(CHOFF, CHLEN):
        pltpu.async_copy(tab_hbm.at[idx_v.at[r, pl.ds(off, ln)]],
                         rows_v.at[p, pl.ds(off, ln)], sems.at[p])
        pltpu.async_copy(w_hbm.at[idx_v.at[r, pl.ds(off, ln)]],
                         wv_v.at[p, pl.ds(off, ln)], sems.at[p])

    def drain(r, p):
      for off, ln in zip(CHOFF, CHLEN):
        pltpu.make_async_copy(tab_hbm.at[idx_v.at[r, pl.ds(off, ln)]],
                              rows_v.at[p, pl.ds(off, ln)], sems.at[p]).wait()
        pltpu.make_async_copy(w_hbm.at[idx_v.at[r, pl.ds(off, ln)]],
                              wv_v.at[p, pl.ds(off, ln)], sems.at[p]).wait()

    def compute(r, p):
      carry = tuple(jnp.zeros((LANES,), jnp.float32) for _ in range(n_d + 1))

      def grp_body(g, c):
        accs, swx = list(c[:-1]), c[-1]
        wraw16 = wv_v[p, pl.ds(g * LANES, LANES)]
        wx16 = 1.0 / (1.0 + jnp.exp(-wraw16))
        for jj in range(LANES):
          t = g * LANES + jj
          wb = jnp.full((LANES,), wx16[jj], jnp.float32)
          for k in range(n_w):
            v = rows_v[p, t, pl.ds(k * LANES, LANES)]
            a = lax.bitcast_convert_type(v << 16, jnp.float32)
            b2 = lax.bitcast_convert_type(v & jnp.int32(-65536), jnp.float32)
            accs[2 * k] = accs[2 * k] + wb * a
            accs[2 * k + 1] = accs[2 * k + 1] + wb * b2
        return tuple(accs) + (swx + wx16,)

      carry = lax.fori_loop(0, n_g, grp_body, carry)

      parts = [carry[-1][i] for i in range(LANES)]
      while len(parts) > 1:
        parts = [a + b for a, b in zip(parts[::2], parts[1::2])]
      denom = jnp.full((LANES,), parts[0] + 1e-16, jnp.float32)
      for k in range(n_d):
        stage_v[r, pl.ds(k * LANES, LANES)] = carry[k] / denom

    n_pair = b_per_w // 2
    fire(0, 0)

    def pair_body(g, _):
      r0 = 2 * g
      fire(r0 + 1, 1)
      drain(r0, 0)
      compute(r0, 0)

      @pl.when(g < n_pair - 1)
      def _():
        fire(r0 + 2, 0)

      drain(r0 + 1, 1)
      compute(r0 + 1, 1)
      return 0

    lax.fori_loop(0, n_pair, pair_body, 0)
    pltpu.sync_copy(stage_v, out_hbm.at[pl.ds(base, b_per_w)])

  return pl.kernel(
      body,
      out_type=jax.ShapeDtypeStruct((B, D), jnp.float32),
      mesh=mesh,
      compiler_params=pltpu.CompilerParams(use_tc_tiling_on_sc=False),
      scratch_types=[
          pltpu.VMEM((b_per_w, L), jnp.int32),
          pltpu.VMEM((2, LTP, D // 2), jnp.int32),
          pltpu.VMEM((2, LTP), jnp.float32),
          pltpu.VMEM((b_per_w, D), jnp.float32),
          pltpu.SemaphoreType.DMA((2,)),
      ],
  )


def _tc_head(B, D, OUT):
  BM = 512

  def body(a_ref, w_ref, b_ref, o_ref):
    o_ref[...] = (
        lax.dot_general(
            a_ref[...], w_ref[...], (((1,), (1,)), ((), ())),
            preferred_element_type=jnp.float32,
        )
        + b_ref[...]
    )

  return pl.pallas_call(
      body,
      grid=(B // BM,),
      in_specs=[
          pl.BlockSpec((BM, D), lambda i: (i, 0)),
          pl.BlockSpec((OUT, D), lambda i: (0, 0)),
          pl.BlockSpec((1, OUT), lambda i: (0, 0)),
      ],
      out_specs=pl.BlockSpec((BM, OUT), lambda i: (i, 0)),
      out_shape=jax.ShapeDtypeStruct((B, OUT), jnp.float32),
  )


def kernel(x, vectors, w, W_out, b_out):
  B, L = x.shape
  V, D = vectors.shape
  OUT = W_out.shape[0]
  vb = (
      vectors.astype(jnp.bfloat16)
      .reshape(V, D // 32, 2, LANES)
      .transpose(0, 1, 3, 2)
      .reshape(V, D // 2, 2)
  )
  tab_packed = lax.bitcast_convert_type(vb, jnp.int32)
  wa = _sc_pool(B, L, V, D)(x, w, tab_packed)
  logits = _tc_head(B, D, OUT)(wa, W_out, b_out.reshape(1, OUT))
  return (logits, wa)
